# Initial kernel scaffold; baseline (speedup 1.0000x reference)
import jax
import jax.numpy as jnp
from jax.experimental import pallas as pl


def kernel(x, edge_index, W_proj, b_proj, W_gcn, b_gcn, W_mlp1, b_mlp1, W_mlp2, b_mlp2, W_mlp3, b_mlp3):
    raise NotImplementedError("write your pallas kernel here")


# speedup vs baseline: 10.2123x; 10.2123x over previous
import functools

import jax
import jax.numpy as jnp
from jax import lax
from jax.experimental import pallas as pl
from jax.experimental.pallas import tpu as pltpu
from jax.experimental.pallas import tpu_sc as plsc

F32 = jnp.float32

N = 10000
E = 320000
H = 128
NPAD = 10240
NC = 2
NS = 16
NWORK = NC * NS
WIN = 128
NW = 80
CHUNK = 8
NCHUNK = NW // CHUNK
EPAD = NWORK * NW * WIN
ROWS_PER_TILE = NPAD // NS
RB = 1000
GRID = N // RB


def _sc_degrees_body(src_hbm, dst_hbm, ones_hbm, zeros_hbm, out_hbm,
                     src_v, dst_v, ones_v, deg_s, deg_d):
    cid = lax.axis_index("c")
    sid = lax.axis_index("s")
    wid = sid * NC + cid
    pltpu.sync_copy(src_hbm.at[wid], src_v)
    pltpu.sync_copy(dst_hbm.at[wid], dst_v)
    pltpu.sync_copy(ones_hbm, ones_v)
    rows = pl.ds(sid * ROWS_PER_TILE, ROWS_PER_TILE)
    pltpu.sync_copy(zeros_hbm.at[rows], deg_s.at[rows])
    pltpu.sync_copy(zeros_hbm.at[rows], deg_d.at[rows])
    plsc.subcore_barrier()

    def body(w, _):
        pltpu.sync_copy(ones_v, deg_s.at[src_v.at[w]], add=True)
        pltpu.sync_copy(ones_v, deg_d.at[dst_v.at[w]], add=True)
        return _

    lax.fori_loop(0, NW, body, None)
    plsc.subcore_barrier()
    pltpu.sync_copy(deg_s.at[rows], out_hbm.at[cid, 0, rows])
    pltpu.sync_copy(deg_d.at[rows], out_hbm.at[cid, 1, rows])


_sc_degrees = pl.kernel(
    _sc_degrees_body,
    out_type=jax.ShapeDtypeStruct((NC, 2, NPAD), F32),
    mesh=plsc.VectorSubcoreMesh(core_axis_name="c", subcore_axis_name="s"),
    scratch_types=[
        pltpu.VMEM((NW, WIN), jnp.int32),
        pltpu.VMEM((NW, WIN), jnp.int32),
        pltpu.VMEM((WIN,), F32),
        pltpu.VMEM_SHARED((NPAD,), F32),
        pltpu.VMEM_SHARED((NPAD,), F32),
    ],
)


def _sc_spmm_body(hs_hbm, src_hbm, dst_hbm, zeros_hbm, out0_hbm, out1_hbm,
                  srcb0, srcb1, dstb0, dstb1, rows0, rows1, agg_sh,
                  gsem0, gsem1, isem_s0, isem_s1, isem_d0, isem_d1):
    cid = lax.axis_index("c")
    sid = lax.axis_index("s")
    wid = sid * NC + cid
    rows = pl.ds(sid * ROWS_PER_TILE, ROWS_PER_TILE)
    pltpu.sync_copy(zeros_hbm.at[rows], agg_sh.at[rows])
    pltpu.sync_copy(src_hbm.at[wid, pl.ds(0, CHUNK)], srcb0)
    pltpu.sync_copy(dst_hbm.at[wid, pl.ds(0, CHUNK)], dstb0)
    pltpu.async_copy(src_hbm.at[wid, pl.ds(CHUNK, CHUNK)], srcb1, isem_s1)
    pltpu.async_copy(dst_hbm.at[wid, pl.ds(CHUNK, CHUNK)], dstb1, isem_d1)
    pltpu.async_copy(hs_hbm.at[srcb0.at[0]], rows0, gsem0)
    plsc.subcore_barrier()

    sbufs = (srcb0, srcb1)
    dbufs = (dstb0, dstb1)
    gbufs = (rows0, rows1)
    gsems = (gsem0, gsem1)
    isems_s = (isem_s0, isem_s1)
    isems_d = (isem_d0, isem_d1)

    def ichunk(c):
        return (src_hbm.at[wid, pl.ds(c * CHUNK, CHUNK)],
                dst_hbm.at[wid, pl.ds(c * CHUNK, CHUNK)])

    def body(s, _):
        for cc in range(2):
            c = 2 * s + cc
            for j in range(CHUNK):
                p = j % 2
                pltpu.make_async_copy(hs_hbm.at[sbufs[cc].at[j]], gbufs[p],
                                      gsems[p]).wait()
                if j + 1 < CHUNK:
                    pltpu.async_copy(hs_hbm.at[sbufs[cc].at[j + 1]],
                                     gbufs[1 - p], gsems[1 - p])
                else:
                    q = 1 - cc
                    nsrc, ndst = ichunk(c + 1)

                    def _cross():
                        pltpu.make_async_copy(nsrc, sbufs[q],
                                              isems_s[q]).wait()
                        pltpu.make_async_copy(ndst, dbufs[q],
                                              isems_d[q]).wait()
                        pltpu.async_copy(hs_hbm.at[sbufs[q].at[0]],
                                         gbufs[1 - p], gsems[1 - p])

                    if cc == 0:
                        _cross()
                    else:
                        pl.when(s < NCHUNK // 2 - 1)(_cross)
                pltpu.sync_copy(gbufs[p], agg_sh.at[dbufs[cc].at[j]], add=True)
                if j + 1 == CHUNK:
                    fsrc, fdst = ichunk(c + 2)

                    def _refill():
                        pltpu.async_copy(fsrc, sbufs[cc], isems_s[cc])
                        pltpu.async_copy(fdst, dbufs[cc], isems_d[cc])

                    pl.when(s < NCHUNK // 2 - 1)(_refill)
        return _

    lax.fori_loop(0, NCHUNK // 2, body, None)
    plsc.subcore_barrier()

    @pl.when(cid == 0)
    def _out0():
        pltpu.sync_copy(agg_sh.at[rows], out0_hbm.at[rows])

    @pl.when(cid == 1)
    def _out1():
        pltpu.sync_copy(agg_sh.at[rows], out1_hbm.at[rows])


_sc_spmm = pl.kernel(
    _sc_spmm_body,
    out_type=[jax.ShapeDtypeStruct((NPAD, H), F32),
              jax.ShapeDtypeStruct((NPAD, H), F32)],
    mesh=plsc.VectorSubcoreMesh(core_axis_name="c", subcore_axis_name="s"),
    scratch_types=[
        pltpu.VMEM((CHUNK, WIN), jnp.int32),
        pltpu.VMEM((CHUNK, WIN), jnp.int32),
        pltpu.VMEM((CHUNK, WIN), jnp.int32),
        pltpu.VMEM((CHUNK, WIN), jnp.int32),
        pltpu.VMEM((WIN, H), F32),
        pltpu.VMEM((WIN, H), F32),
        pltpu.VMEM_SHARED((NPAD, H), F32),
        pltpu.SemaphoreType.DMA,
        pltpu.SemaphoreType.DMA,
        pltpu.SemaphoreType.DMA,
        pltpu.SemaphoreType.DMA,
        pltpu.SemaphoreType.DMA,
        pltpu.SemaphoreType.DMA,
    ],
)


def _tc_proj_body(x_ref, w_ref, b_ref, deg_ref, h_ref, hs_ref, ns_ref, nd_ref):
    dsrc = deg_ref[0, 0] + deg_ref[1, 0]
    ddst = deg_ref[0, 1] + deg_ref[1, 1]
    ns = lax.rsqrt(jnp.maximum(dsrc, 1.0))
    nd = lax.rsqrt(jnp.maximum(ddst, 1.0))
    h = jnp.dot(x_ref[...], w_ref[...], preferred_element_type=F32) + b_ref[...]
    h_ref[...] = h
    hs_ref[...] = h * ns
    ns_ref[...] = ns
    nd_ref[...] = nd


_tc_proj = pl.pallas_call(
    _tc_proj_body,
    grid=(GRID,),
    in_specs=[
        pl.BlockSpec((RB, H), lambda i: (i, 0)),
        pl.BlockSpec((H, H), lambda i: (0, 0)),
        pl.BlockSpec((1, H), lambda i: (0, 0)),
        pl.BlockSpec((NC, 2, RB, 1), lambda i: (0, 0, i, 0)),
    ],
    out_specs=[
        pl.BlockSpec((RB, H), lambda i: (i, 0)),
        pl.BlockSpec((RB, H), lambda i: (i, 0)),
        pl.BlockSpec((RB, 1), lambda i: (i, 0)),
        pl.BlockSpec((RB, 1), lambda i: (i, 0)),
    ],
    out_shape=[
        jax.ShapeDtypeStruct((N, H), F32),
        jax.ShapeDtypeStruct((N, H), F32),
        jax.ShapeDtypeStruct((N, 1), F32),
        jax.ShapeDtypeStruct((N, 1), F32),
    ],
)


def _make_tc_layer(act):
    def body(h_ref, a0_ref, a1_ref, nd_ref, ns_ref, w_ref, b_ref,
             hn_ref, hs_ref):
        aggn = (a0_ref[...] + a1_ref[...]) * nd_ref[...]
        out = jnp.dot(aggn, w_ref[...], preferred_element_type=F32) + b_ref[...]
        if act:
            out = jnp.maximum(out, 0.0)
        hnew = jnp.maximum(h_ref[...] + out, 0.0)
        hn_ref[...] = hnew
        hs_ref[...] = hnew * ns_ref[...]

    return pl.pallas_call(
        body,
        grid=(GRID,),
        in_specs=[
            pl.BlockSpec((RB, H), lambda i: (i, 0)),
            pl.BlockSpec((RB, H), lambda i: (i, 0)),
            pl.BlockSpec((RB, H), lambda i: (i, 0)),
            pl.BlockSpec((RB, 1), lambda i: (i, 0)),
            pl.BlockSpec((RB, 1), lambda i: (i, 0)),
            pl.BlockSpec((H, H), lambda i: (0, 0)),
            pl.BlockSpec((1, H), lambda i: (0, 0)),
        ],
        out_specs=[
            pl.BlockSpec((RB, H), lambda i: (i, 0)),
            pl.BlockSpec((RB, H), lambda i: (i, 0)),
        ],
        out_shape=[
            jax.ShapeDtypeStruct((N, H), F32),
            jax.ShapeDtypeStruct((N, H), F32),
        ],
    )


_tc_layer_act = _make_tc_layer(True)


def _tc_tail_body(h_ref, a0_ref, a1_ref, nd_ref, w_ref, b_ref,
                  wm1_ref, bm1_ref, wm2_ref, bm2_ref, wm3_ref, bm3_ref,
                  out_ref, acc_ref):
    i = pl.program_id(0)
    aggn = (a0_ref[...] + a1_ref[...]) * nd_ref[...]
    out = jnp.dot(aggn, w_ref[...], preferred_element_type=F32) + b_ref[...]
    hnew = jnp.maximum(h_ref[...] + out, 0.0)
    psum = jnp.sum(hnew, axis=0, keepdims=True)

    @pl.when(i == 0)
    def _init():
        acc_ref[...] = psum

    @pl.when(i > 0)
    def _acc():
        acc_ref[...] += psum

    @pl.when(i == pl.num_programs(0) - 1)
    def _head():
        hg = acc_ref[...]
        z = jnp.dot(hg, wm1_ref[...], preferred_element_type=F32) + bm1_ref[...]
        z = jnp.maximum(z, 0.0)
        z = jnp.dot(z, wm2_ref[...], preferred_element_type=F32) + bm2_ref[...]
        z = jnp.maximum(z, 0.0)
        z = jnp.dot(z, wm3_ref[...], preferred_element_type=F32) + bm3_ref[...]
        out_ref[...] = z


def _make_tc_tail(mlp):
    return pl.pallas_call(
        _tc_tail_body,
        grid=(GRID,),
        in_specs=[
            pl.BlockSpec((RB, H), lambda i: (i, 0)),
            pl.BlockSpec((RB, H), lambda i: (i, 0)),
            pl.BlockSpec((RB, H), lambda i: (i, 0)),
            pl.BlockSpec((RB, 1), lambda i: (i, 0)),
            pl.BlockSpec((H, H), lambda i: (0, 0)),
            pl.BlockSpec((1, H), lambda i: (0, 0)),
            pl.BlockSpec((H, mlp), lambda i: (0, 0)),
            pl.BlockSpec((1, mlp), lambda i: (0, 0)),
            pl.BlockSpec((mlp, mlp), lambda i: (0, 0)),
            pl.BlockSpec((1, mlp), lambda i: (0, 0)),
            pl.BlockSpec((mlp, mlp), lambda i: (0, 0)),
            pl.BlockSpec((1, mlp), lambda i: (0, 0)),
        ],
        out_specs=pl.BlockSpec((1, mlp), lambda i: (0, 0)),
        out_shape=jax.ShapeDtypeStruct((1, mlp), F32),
        scratch_shapes=[pltpu.VMEM((1, H), F32)],
    )


def kernel(x, edge_index, W_proj, b_proj, W_gcn, b_gcn,
           W_mlp1, b_mlp1, W_mlp2, b_mlp2, W_mlp3, b_mlp3):
    mlp = W_mlp1.shape[1]
    src = edge_index[0]
    dst = edge_index[1]
    npad_e = EPAD - E
    spread = (jnp.arange(npad_e, dtype=jnp.int32) % 128)

    src_p = jnp.concatenate([src, spread]).reshape(NWORK, NW, WIN)
    dst_p = jnp.concatenate([dst, N + spread]).reshape(NWORK, NW, WIN)
    src_pd = jnp.concatenate([src, N + spread]).reshape(NWORK, NW, WIN)
    dst_pd = dst_p

    ones_w = jnp.ones((WIN,), F32)
    zeros_1d = jnp.zeros((NPAD,), F32)
    zeros_2d = jnp.zeros((NPAD, H), F32)

    degp = _sc_degrees(src_pd, dst_pd, ones_w, zeros_1d)
    degp = degp[:, :, :N].reshape(NC, 2, N, 1)

    b_proj2 = b_proj.reshape(1, H)
    h, hs, ns, nd = _tc_proj(x, W_proj, b_proj2, degp)

    for i in range(W_gcn.shape[0] - 1):
        a0, a1 = _sc_spmm(hs, src_p, dst_p, zeros_2d)
        h, hs = _tc_layer_act(h, a0[:N], a1[:N], nd, ns,
                              W_gcn[i], b_gcn[i].reshape(1, H))

    a0, a1 = _sc_spmm(hs, src_p, dst_p, zeros_2d)
    tail = _make_tc_tail(mlp)
    hg = tail(h, a0[:N], a1[:N], nd,
              W_gcn[-1], b_gcn[-1].reshape(1, H),
              W_mlp1, b_mlp1.reshape(1, mlp),
              W_mlp2, b_mlp2.reshape(1, mlp),
              W_mlp3, b_mlp3.reshape(1, mlp))
    return hg


# speedup vs baseline: 10.6829x; 1.0461x over previous
import functools

import jax
import jax.numpy as jnp
from jax import lax
from jax.experimental import pallas as pl
from jax.experimental.pallas import tpu as pltpu
from jax.experimental.pallas import tpu_sc as plsc

F32 = jnp.float32

N = 10000
E = 320000
H = 128
NPAD = 10240
NC = 2
NS = 16
NWORK = NC * NS
WIN = 128
NW = 80
CHUNK = 8
NCHUNK = NW // CHUNK
EPAD = NWORK * NW * WIN
ROWS_PER_TILE = NPAD // NS
RB = 1000
GRID = N // RB


def _sc_degrees_body(src_hbm, dst_hbm, ones_hbm, zeros_hbm, out_hbm,
                     src_v, dst_v, ones_v, deg_s, deg_d):
    cid = lax.axis_index("c")
    sid = lax.axis_index("s")
    wid = sid * NC + cid
    pltpu.sync_copy(src_hbm.at[wid], src_v)
    pltpu.sync_copy(dst_hbm.at[wid], dst_v)
    pltpu.sync_copy(ones_hbm, ones_v)
    rows = pl.ds(sid * ROWS_PER_TILE, ROWS_PER_TILE)
    pltpu.sync_copy(zeros_hbm.at[rows], deg_s.at[rows])
    pltpu.sync_copy(zeros_hbm.at[rows], deg_d.at[rows])
    plsc.subcore_barrier()

    def body(w, _):
        pltpu.sync_copy(ones_v, deg_s.at[src_v.at[w]], add=True)
        pltpu.sync_copy(ones_v, deg_d.at[dst_v.at[w]], add=True)
        return _

    lax.fori_loop(0, NW, body, None)
    plsc.subcore_barrier()
    pltpu.sync_copy(deg_s.at[rows], out_hbm.at[cid, 0, rows])
    pltpu.sync_copy(deg_d.at[rows], out_hbm.at[cid, 1, rows])


_sc_degrees = pl.kernel(
    _sc_degrees_body,
    out_type=jax.ShapeDtypeStruct((NC, 2, NPAD), F32),
    mesh=plsc.VectorSubcoreMesh(core_axis_name="c", subcore_axis_name="s"),
    scratch_types=[
        pltpu.VMEM((NW, WIN), jnp.int32),
        pltpu.VMEM((NW, WIN), jnp.int32),
        pltpu.VMEM((WIN,), F32),
        pltpu.VMEM_SHARED((NPAD,), F32),
        pltpu.VMEM_SHARED((NPAD,), F32),
    ],
)


def _sc_spmm_body(hs_hbm, src_hbm, dst_hbm, zeros_hbm, out0_hbm, out1_hbm,
                  srcb0, srcb1, dstb0, dstb1, rows0, rows1, agg_sh,
                  gsem0, gsem1, isem_s0, isem_s1, isem_d0, isem_d1):
    cid = lax.axis_index("c")
    sid = lax.axis_index("s")
    wid = sid * NC + cid
    rows = pl.ds(sid * ROWS_PER_TILE, ROWS_PER_TILE)
    pltpu.sync_copy(zeros_hbm.at[rows], agg_sh.at[rows])
    pltpu.sync_copy(src_hbm.at[wid, pl.ds(0, CHUNK)], srcb0)
    pltpu.sync_copy(dst_hbm.at[wid, pl.ds(0, CHUNK)], dstb0)
    pltpu.async_copy(src_hbm.at[wid, pl.ds(CHUNK, CHUNK)], srcb1, isem_s1)
    pltpu.async_copy(dst_hbm.at[wid, pl.ds(CHUNK, CHUNK)], dstb1, isem_d1)
    pltpu.async_copy(hs_hbm.at[srcb0.at[0]], rows0, gsem0)
    plsc.subcore_barrier()

    sbufs = (srcb0, srcb1)
    dbufs = (dstb0, dstb1)
    gbufs = (rows0, rows1)
    gsems = (gsem0, gsem1)
    isems_s = (isem_s0, isem_s1)
    isems_d = (isem_d0, isem_d1)

    def ichunk(c):
        return (src_hbm.at[wid, pl.ds(c * CHUNK, CHUNK)],
                dst_hbm.at[wid, pl.ds(c * CHUNK, CHUNK)])

    def body(s, _):
        for cc in range(2):
            c = 2 * s + cc
            for j in range(CHUNK):
                p = j % 2
                pltpu.make_async_copy(hs_hbm.at[sbufs[cc].at[j]], gbufs[p],
                                      gsems[p]).wait()
                if j + 1 < CHUNK:
                    pltpu.async_copy(hs_hbm.at[sbufs[cc].at[j + 1]],
                                     gbufs[1 - p], gsems[1 - p])
                else:
                    q = 1 - cc
                    nsrc, ndst = ichunk(c + 1)

                    def _cross():
                        pltpu.make_async_copy(nsrc, sbufs[q],
                                              isems_s[q]).wait()
                        pltpu.make_async_copy(ndst, dbufs[q],
                                              isems_d[q]).wait()
                        pltpu.async_copy(hs_hbm.at[sbufs[q].at[0]],
                                         gbufs[1 - p], gsems[1 - p])

                    if cc == 0:
                        _cross()
                    else:
                        pl.when(s < NCHUNK // 2 - 1)(_cross)
                pltpu.sync_copy(gbufs[p], agg_sh.at[dbufs[cc].at[j]], add=True)
                if j + 1 == CHUNK:
                    fsrc, fdst = ichunk(c + 2)

                    def _refill():
                        pltpu.async_copy(fsrc, sbufs[cc], isems_s[cc])
                        pltpu.async_copy(fdst, dbufs[cc], isems_d[cc])

                    pl.when(s < NCHUNK // 2 - 1)(_refill)
        return _

    lax.fori_loop(0, NCHUNK // 2, body, None)
    plsc.subcore_barrier()

    @pl.when(cid == 0)
    def _out0():
        pltpu.sync_copy(agg_sh.at[rows], out0_hbm.at[rows])

    @pl.when(cid == 1)
    def _out1():
        pltpu.sync_copy(agg_sh.at[rows], out1_hbm.at[rows])


_sc_spmm = pl.kernel(
    _sc_spmm_body,
    out_type=[jax.ShapeDtypeStruct((NPAD, H), F32),
              jax.ShapeDtypeStruct((NPAD, H), F32)],
    mesh=plsc.VectorSubcoreMesh(core_axis_name="c", subcore_axis_name="s"),
    scratch_types=[
        pltpu.VMEM((CHUNK, WIN), jnp.int32),
        pltpu.VMEM((CHUNK, WIN), jnp.int32),
        pltpu.VMEM((CHUNK, WIN), jnp.int32),
        pltpu.VMEM((CHUNK, WIN), jnp.int32),
        pltpu.VMEM((WIN, H), F32),
        pltpu.VMEM((WIN, H), F32),
        pltpu.VMEM_SHARED((NPAD, H), F32),
        pltpu.SemaphoreType.DMA,
        pltpu.SemaphoreType.DMA,
        pltpu.SemaphoreType.DMA,
        pltpu.SemaphoreType.DMA,
        pltpu.SemaphoreType.DMA,
        pltpu.SemaphoreType.DMA,
    ],
)


def _tc_proj_body(x_ref, w_ref, b_ref, deg_ref, h_ref, hs_ref, ns_ref, nd_ref):
    dsrc = deg_ref[0, 0] + deg_ref[1, 0]
    ddst = deg_ref[0, 1] + deg_ref[1, 1]
    ns = lax.rsqrt(jnp.maximum(dsrc, 1.0))
    nd = lax.rsqrt(jnp.maximum(ddst, 1.0))
    h = jnp.dot(x_ref[...], w_ref[...], preferred_element_type=F32) + b_ref[...]
    h_ref[...] = h
    hs_ref[...] = h * ns
    ns_ref[...] = ns
    nd_ref[...] = nd


_tc_proj = pl.pallas_call(
    _tc_proj_body,
    grid=(GRID,),
    in_specs=[
        pl.BlockSpec((RB, H), lambda i: (i, 0)),
        pl.BlockSpec((H, H), lambda i: (0, 0)),
        pl.BlockSpec((1, H), lambda i: (0, 0)),
        pl.BlockSpec((NC, 2, RB, 1), lambda i: (0, 0, i, 0)),
    ],
    out_specs=[
        pl.BlockSpec((RB, H), lambda i: (i, 0)),
        pl.BlockSpec((RB, H), lambda i: (i, 0)),
        pl.BlockSpec((RB, 1), lambda i: (i, 0)),
        pl.BlockSpec((RB, 1), lambda i: (i, 0)),
    ],
    out_shape=[
        jax.ShapeDtypeStruct((N, H), F32),
        jax.ShapeDtypeStruct((N, H), F32),
        jax.ShapeDtypeStruct((N, 1), F32),
        jax.ShapeDtypeStruct((N, 1), F32),
    ],
)


def _make_tc_layer(act):
    def body(h_ref, a0_ref, a1_ref, nd_ref, ns_ref, w_ref, b_ref,
             hn_ref, hs_ref):
        aggn = (a0_ref[...] + a1_ref[...]) * nd_ref[...]
        out = jnp.dot(aggn, w_ref[...], preferred_element_type=F32) + b_ref[...]
        if act:
            out = jnp.maximum(out, 0.0)
        hnew = jnp.maximum(h_ref[...] + out, 0.0)
        hn_ref[...] = hnew
        hs_ref[...] = hnew * ns_ref[...]

    return pl.pallas_call(
        body,
        grid=(GRID,),
        in_specs=[
            pl.BlockSpec((RB, H), lambda i: (i, 0)),
            pl.BlockSpec((RB, H), lambda i: (i, 0)),
            pl.BlockSpec((RB, H), lambda i: (i, 0)),
            pl.BlockSpec((RB, 1), lambda i: (i, 0)),
            pl.BlockSpec((RB, 1), lambda i: (i, 0)),
            pl.BlockSpec((H, H), lambda i: (0, 0)),
            pl.BlockSpec((1, H), lambda i: (0, 0)),
        ],
        out_specs=[
            pl.BlockSpec((RB, H), lambda i: (i, 0)),
            pl.BlockSpec((RB, H), lambda i: (i, 0)),
        ],
        out_shape=[
            jax.ShapeDtypeStruct((N, H), F32),
            jax.ShapeDtypeStruct((N, H), F32),
        ],
    )


_tc_layer_act = _make_tc_layer(True)


def _tc_tail_body(h_ref, a0_ref, a1_ref, nd_ref, w_ref, b_ref,
                  wm1_ref, bm1_ref, wm2_ref, bm2_ref, wm3_ref, bm3_ref,
                  out_ref, acc_ref):
    i = pl.program_id(0)
    aggn = (a0_ref[...] + a1_ref[...]) * nd_ref[...]
    out = jnp.dot(aggn, w_ref[...], preferred_element_type=F32) + b_ref[...]
    hnew = jnp.maximum(h_ref[...] + out, 0.0)
    psum = jnp.sum(hnew, axis=0, keepdims=True)

    @pl.when(i == 0)
    def _init():
        acc_ref[...] = psum

    @pl.when(i > 0)
    def _acc():
        acc_ref[...] += psum

    @pl.when(i == pl.num_programs(0) - 1)
    def _head():
        hg = acc_ref[...]
        z = jnp.dot(hg, wm1_ref[...], preferred_element_type=F32) + bm1_ref[...]
        z = jnp.maximum(z, 0.0)
        z = jnp.dot(z, wm2_ref[...], preferred_element_type=F32) + bm2_ref[...]
        z = jnp.maximum(z, 0.0)
        z = jnp.dot(z, wm3_ref[...], preferred_element_type=F32) + bm3_ref[...]
        out_ref[...] = z


def _make_tc_tail(mlp):
    return pl.pallas_call(
        _tc_tail_body,
        grid=(GRID,),
        in_specs=[
            pl.BlockSpec((RB, H), lambda i: (i, 0)),
            pl.BlockSpec((RB, H), lambda i: (i, 0)),
            pl.BlockSpec((RB, H), lambda i: (i, 0)),
            pl.BlockSpec((RB, 1), lambda i: (i, 0)),
            pl.BlockSpec((H, H), lambda i: (0, 0)),
            pl.BlockSpec((1, H), lambda i: (0, 0)),
            pl.BlockSpec((H, mlp), lambda i: (0, 0)),
            pl.BlockSpec((1, mlp), lambda i: (0, 0)),
            pl.BlockSpec((mlp, mlp), lambda i: (0, 0)),
            pl.BlockSpec((1, mlp), lambda i: (0, 0)),
            pl.BlockSpec((mlp, mlp), lambda i: (0, 0)),
            pl.BlockSpec((1, mlp), lambda i: (0, 0)),
        ],
        out_specs=pl.BlockSpec((1, mlp), lambda i: (0, 0)),
        out_shape=jax.ShapeDtypeStruct((1, mlp), F32),
        scratch_shapes=[pltpu.VMEM((1, H), F32)],
    )


def kernel(x, edge_index, W_proj, b_proj, W_gcn, b_gcn,
           W_mlp1, b_mlp1, W_mlp2, b_mlp2, W_mlp3, b_mlp3):
    mlp = W_mlp1.shape[1]
    src = edge_index[0]
    dst = edge_index[1]
    npad_e = EPAD - E
    spread = (jnp.arange(npad_e, dtype=jnp.int32) % 128)

    src_p = jnp.concatenate([src, spread]).reshape(NWORK, NW, WIN)
    dst_p = jnp.concatenate([dst, N + spread]).reshape(NWORK, NW, WIN)
    src_pd = jnp.concatenate([src, N + spread]).reshape(NWORK, NW, WIN)
    dst_pd = dst_p

    ones_w = jnp.ones((WIN,), F32)
    zeros_1d = jnp.zeros((NPAD,), F32)
    zeros_2d = jnp.zeros((NPAD, H), F32)

    degp = _sc_degrees(src_pd, dst_pd, ones_w, zeros_1d)
    degp = degp.reshape(NC, 2, NPAD, 1)

    b_proj2 = b_proj.reshape(1, H)
    h, hs, ns, nd = _tc_proj(x, W_proj, b_proj2, degp)

    for i in range(W_gcn.shape[0] - 1):
        a0, a1 = _sc_spmm(hs, src_p, dst_p, zeros_2d)
        h, hs = _tc_layer_act(h, a0, a1, nd, ns,
                              W_gcn[i], b_gcn[i].reshape(1, H))

    a0, a1 = _sc_spmm(hs, src_p, dst_p, zeros_2d)
    tail = _make_tc_tail(mlp)
    hg = tail(h, a0, a1, nd,
              W_gcn[-1], b_gcn[-1].reshape(1, H),
              W_mlp1, b_mlp1.reshape(1, mlp),
              W_mlp2, b_mlp2.reshape(1, mlp),
              W_mlp3, b_mlp3.reshape(1, mlp))
    return hg
